# baseline (speedup 1.0000x reference)
import functools
import math

import jax
import jax.numpy as jnp
from jax import lax
from jax.experimental import pallas as pl
from jax.experimental.pallas import tpu as pltpu
from jax.experimental.pallas import tpu_sc as plsc


def _positional_encoding(seq, emb):
    positions = jnp.arange(0, seq, dtype=jnp.float32)[:, None]
    div_term = 10000.0 ** (jnp.arange(0, emb, 2, dtype=jnp.float32) / emb)
    pe = jnp.zeros((seq, emb), dtype=jnp.float32)
    pe = pe.at[:, 0::2].set(jnp.sin(positions / div_term))
    pe = pe.at[:, 1::2].set(jnp.cos(positions / div_term))
    return pe


@functools.partial(jax.jit, static_argnames=("n", "d", "seq"))
def _emb_call(idx3, table, pe, *, n, d, seq):
    info = plsc.get_sparse_core_info()
    nc, ns, lanes = info.num_cores, info.num_subcores, info.num_lanes
    nw = nc * ns
    per_w = n // nw
    rows_pc = 128
    n_chunks = per_w // rows_pc

    mesh = plsc.VectorSubcoreMesh(core_axis_name="c", subcore_axis_name="s")

    @functools.partial(
        pl.kernel,
        mesh=mesh,
        out_type=jax.ShapeDtypeStruct((n, d), jnp.float32),
        scratch_types=[
            pltpu.VMEM((n_chunks, rows_pc), jnp.int32),
            pltpu.VMEM((rows_pc, d), jnp.float32),
            pltpu.VMEM((rows_pc, d), jnp.float32),
            pltpu.SemaphoreType.DMA,
        ],
    )
    def emb_kernel(idx_hbm, table_hbm, pe_hbm, out_hbm, idx_v, rows_v, pe_v, sem):
        wid = lax.axis_index("s") * nc + lax.axis_index("c")
        base = wid * per_w
        pe_base = lax.rem(base, seq)
        pltpu.sync_copy(idx_hbm.at[wid], idx_v)

        scale = float(math.sqrt(d))
        for c in range(n_chunks):
            row0 = c * rows_pc
            gather = pltpu.async_copy(
                table_hbm.at[idx_v.at[c]], rows_v, sem
            )
            pltpu.sync_copy(pe_hbm.at[pl.ds(pe_base + row0, rows_pc)], pe_v)
            gather.wait()

            def body(i, _):
                for j in range(d // lanes):
                    sl = pl.ds(j * lanes, lanes)
                    rows_v[i, sl] = rows_v[i, sl] * scale + pe_v[i, sl]
                return 0

            lax.fori_loop(0, rows_pc, body, 0)
            pltpu.sync_copy(rows_v, out_hbm.at[pl.ds(base + row0, rows_pc)])

    return emb_kernel(idx3, table, pe)


def kernel(x, table):
    b, s = x.shape
    v, d = table.shape
    n = b * s
    nw = 32
    idx3 = x.reshape(nw, n // nw // 128, 128)
    pe = _positional_encoding(s, d)
    out = _emb_call(idx3, table, pe, n=n, d=d, seq=s)
    return out.reshape(b, s, d)


# speedup vs baseline: 1.6789x; 1.6789x over previous
import functools
import math

import numpy as np
import jax
import jax.numpy as jnp
from jax import lax
from jax.experimental import pallas as pl
from jax.experimental.pallas import tpu as pltpu
from jax.experimental.pallas import tpu_sc as plsc

_EMB = 128
_SEQ = 4096


@functools.lru_cache(maxsize=None)
def _positional_encoding(seq, emb):
    positions = np.arange(0, seq, dtype=np.float32)[:, None]
    div_term = 10000.0 ** (np.arange(0, emb, 2, dtype=np.float32) / emb)
    pe = np.zeros((seq, emb), dtype=np.float32)
    pe[:, 0::2] = np.sin(positions / div_term)
    pe[:, 1::2] = np.cos(positions / div_term)
    return pe


def kernel(x, table):
    b, s = x.shape
    v, d = table.shape
    n = b * s
    scale = float(math.sqrt(d))

    info = plsc.get_sparse_core_info()
    nc, ns, lanes = info.num_cores, info.num_subcores, info.num_lanes
    nw = nc * ns
    blk = s // nw
    assert blk <= 128

    pe = jnp.asarray(_positional_encoding(s, d))
    idx3 = x.reshape(b, nw, blk).transpose(1, 0, 2)

    mesh = plsc.VectorSubcoreMesh(core_axis_name="c", subcore_axis_name="s")

    @functools.partial(
        pl.kernel,
        mesh=mesh,
        out_type=jax.ShapeDtypeStruct((n, d), jnp.float32),
        scratch_types=[
            pltpu.VMEM((b, blk), jnp.int32),
            pltpu.VMEM((blk, d), jnp.float32),
        ]
        + [pltpu.VMEM((blk, d), jnp.float32) for _ in range(b)]
        + [
            pltpu.SemaphoreType.DMA,
            pltpu.SemaphoreType.DMA,
            pltpu.SemaphoreType.DMA,
        ],
    )
    def emb_kernel(idx_hbm, table_hbm, pe_hbm, out_hbm, idx_v, pe_v, *rest):
        rows = rest[:b]
        gsem, psem, osem = rest[b:]
        wid = lax.axis_index("s") * nc + lax.axis_index("c")
        pos0 = wid * blk

        pltpu.sync_copy(idx_hbm.at[wid], idx_v)
        pe_cp = pltpu.async_copy(pe_hbm.at[pl.ds(pos0, blk)], pe_v, psem)
        gathers = [
            pltpu.async_copy(table_hbm.at[idx_v.at[j]], rows[j], gsem)
            for j in range(b)
        ]
        pe_cp.wait()

        stores = []
        for j in range(b):
            gathers[j].wait()
            rv = rows[j]

            def body(i, _):
                for k in range(d // lanes):
                    sl = pl.ds(k * lanes, lanes)
                    rv[i, sl] = rv[i, sl] * scale + pe_v[i, sl]
                return 0

            lax.fori_loop(0, blk, body, 0)
            stores.append(
                pltpu.async_copy(
                    rv, out_hbm.at[pl.ds(j * s + pos0, blk)], osem
                )
            )
        for st in stores:
            st.wait()

    out = emb_kernel(idx3, table, pe)
    return out.reshape(b, s, d)
